# Initial kernel scaffold; baseline (speedup 1.0000x reference)
import jax
import jax.numpy as jnp
from jax.experimental import pallas as pl


def kernel(xA, edge_indexA, edge_attrA, batchA, xB, edge_indexB, edge_attrB, batchB, context, params):
    raise NotImplementedError("write your pallas kernel here")


# speedup vs baseline: 3.5435x; 3.5435x over previous
import functools

import jax
import jax.numpy as jnp
from jax import lax
from jax.experimental import pallas as pl
from jax.experimental.pallas import tpu as pltpu
from jax.experimental.pallas import tpu_sc as plsc

N = 10000
E = 160000
B = 64
EMB = 128
NUM_BOND = 6
NUM_ATOM = 120
NUM_LAYER = 5

N2 = 2 * N
E2 = 2 * E
R = 2000
NB = N2 // R
HALF = EMB // 2

SC_SUB = 16
K = 128
PH = 2
HN = N // PH
PT = HN // SC_SUB // 8 * 8
PT_LAST = HN - 15 * PT
ACC_ROWS = HN + 8
EPAD = E + K


def _sc_aggregate(T2, gidx, dst, starts, zrows):
    mesh = plsc.VectorSubcoreMesh(core_axis_name="c", subcore_axis_name="s")

    @functools.partial(
        pl.kernel,
        out_type=jax.ShapeDtypeStruct((2, N, EMB), jnp.float32),
        mesh=mesh,
        scratch_types=[
            pltpu.VMEM((K,), jnp.int32),
            pltpu.VMEM((K,), jnp.int32),
            pltpu.VMEM((K,), jnp.int32),
            pltpu.VMEM((K, EMB), jnp.float32),
            pltpu.VMEM((2, 16), jnp.int32),
            pltpu.VMEM_SHARED((ACC_ROWS, EMB), jnp.float32),
            pltpu.SemaphoreType.DMA,
        ],
    )
    def agg(t_hbm, gidx_hbm, dst_hbm, st_hbm, z_hbm, out_hbm,
            gi_v, di_v, dl_v, msg_v, st_v, acc_sh, sem):
        c = lax.axis_index("c")
        s = lax.axis_index("s")

        pltpu.sync_copy(st_hbm, st_v)
        svec = st_v[c, 0:16]
        trash = HN + (jnp.arange(16, dtype=jnp.int32) & 7)
        row0 = s * PT

        for p in range(PH):
            @pl.when(s < SC_SUB - 1)
            def _():
                pltpu.sync_copy(z_hbm.at[pl.ds(0, PT)],
                                acc_sh.at[pl.ds(row0, PT)])

            @pl.when(s == SC_SUB - 1)
            def _():
                pltpu.sync_copy(z_hbm, acc_sh.at[pl.ds(15 * PT, PT_LAST + 8)])

            plsc.subcore_barrier()

            s0 = svec[p]
            s1 = svec[p + 1]
            lo = (s0 // 8) * 8
            nch = (s1 - lo + K - 1) // K
            nmy = (nch - s + SC_SUB - 1) // SC_SUB
            ebase = c * EPAD + lo
            pbase = p * HN

            def body(t, _):
                off = ebase + (s + t * SC_SUB) * K
                pltpu.sync_copy(gidx_hbm.at[pl.ds(off, K)], gi_v)
                pltpu.sync_copy(dst_hbm.at[pl.ds(off, K)], di_v)
                gcp = pltpu.async_copy(t_hbm.at[gi_v], msg_v, sem)
                for j in range(K // 16):
                    d = di_v[j * 16:(j + 1) * 16] - pbase
                    ok = (d >= 0) & (d < HN)
                    dl_v[j * 16:(j + 1) * 16] = jnp.where(ok, d, trash)
                gcp.wait()
                pltpu.sync_copy(msg_v, acc_sh.at[dl_v], add=True)
                return 0

            lax.fori_loop(0, nmy, body, 0)
            plsc.subcore_barrier()

            @pl.when(s < SC_SUB - 1)
            def _():
                pltpu.sync_copy(acc_sh.at[pl.ds(row0, PT)],
                                out_hbm.at[c].at[pl.ds(pbase + row0, PT)])

            @pl.when(s == SC_SUB - 1)
            def _():
                pltpu.sync_copy(acc_sh.at[pl.ds(15 * PT, PT_LAST)],
                                out_hbm.at[c].at[pl.ds(pbase + 15 * PT, PT_LAST)])

            plsc.subcore_barrier()

    return agg(T2, gidx, dst, starts, zrows)


def _embed(xf, x_emb, e_emb):
    def body(x_ref, xe_ref, ee_ref, h_ref, t_ref):
        ids = x_ref[...]
        io = lax.broadcasted_iota(jnp.int32, (R, NUM_ATOM), 1).astype(jnp.float32)
        oh = (io == ids).astype(jnp.float32)
        h = jnp.dot(oh, xe_ref[...], preferred_element_type=jnp.float32)
        h_ref[...] = h
        for b in range(NUM_BOND):
            t_ref[b] = jnp.maximum(h + ee_ref[b, :][None, :], 0.0)

    return pl.pallas_call(
        body,
        grid=(NB,),
        in_specs=[
            pl.BlockSpec((R, 1), lambda i: (i, 0)),
            pl.BlockSpec((NUM_ATOM, EMB), lambda i: (0, 0)),
            pl.BlockSpec((NUM_BOND, EMB), lambda i: (0, 0)),
        ],
        out_specs=[
            pl.BlockSpec((R, EMB), lambda i: (i, 0)),
            pl.BlockSpec((NUM_BOND, R, EMB), lambda i: (0, i, 0)),
        ],
        out_shape=[
            jax.ShapeDtypeStruct((N2, EMB), jnp.float32),
            jax.ShapeDtypeStruct((NUM_BOND, N2, EMB), jnp.float32),
        ],
    )(xf, x_emb, e_emb)


def _mlp(h, aggr, W1, b1, W2, b2):
    def body(h_ref, a_ref, w1_ref, b1_ref, w2_ref, b2_ref, u_ref, st_ref):
        i = pl.program_id(0)
        z = h_ref[...] + a_ref[...]
        t = jnp.maximum(jnp.dot(z, w1_ref[...], preferred_element_type=jnp.float32)
                        + b1_ref[...], 0.0)
        u = jnp.dot(t, w2_ref[...], preferred_element_type=jnp.float32) + b2_ref[...]
        u_ref[...] = u

        @pl.when(i % (NB // 2) == 0)
        def _():
            st_ref[...] = jnp.zeros((1, 8, EMB), jnp.float32)

        st_ref[0, 0:1, :] += jnp.sum(u, axis=0, keepdims=True)
        st_ref[0, 1:2, :] += jnp.sum(u * u, axis=0, keepdims=True)

    return pl.pallas_call(
        body,
        grid=(NB,),
        in_specs=[
            pl.BlockSpec((R, EMB), lambda i: (i, 0)),
            pl.BlockSpec((R, EMB), lambda i: (i, 0)),
            pl.BlockSpec((EMB, 2 * EMB), lambda i: (0, 0)),
            pl.BlockSpec((1, 2 * EMB), lambda i: (0, 0)),
            pl.BlockSpec((2 * EMB, EMB), lambda i: (0, 0)),
            pl.BlockSpec((1, EMB), lambda i: (0, 0)),
        ],
        out_specs=[
            pl.BlockSpec((R, EMB), lambda i: (i, 0)),
            pl.BlockSpec((1, 8, EMB), lambda i: (i // (NB // 2), 0, 0)),
        ],
        out_shape=[
            jax.ShapeDtypeStruct((N2, EMB), jnp.float32),
            jax.ShapeDtypeStruct((2, 8, EMB), jnp.float32),
        ],
    )(h, aggr, W1, b1, W2, b2)


def _norm(u, stats, gamma, beta, e_emb, last):
    def body(u_ref, st_ref, g_ref, be_ref, ee_ref, h_ref, *maybe_t):
        i = pl.program_id(0)
        is_a = (i < NB // 2)
        s = jnp.where(is_a, st_ref[0, 0:1, :], st_ref[1, 0:1, :])
        s2 = jnp.where(is_a, st_ref[0, 1:2, :], st_ref[1, 1:2, :])
        mu = s * (1.0 / N)
        var = s2 * (1.0 / N) - mu * mu
        inv = lax.rsqrt(var + 1e-5)
        hn = g_ref[...] * (u_ref[...] - mu) * inv + be_ref[...]
        if not last:
            hn = jnp.maximum(hn, 0.0)
        h_ref[...] = hn
        if not last:
            t_ref = maybe_t[0]
            for b in range(NUM_BOND):
                t_ref[b] = jnp.maximum(hn + ee_ref[b, :][None, :], 0.0)

    out_specs = [pl.BlockSpec((R, EMB), lambda i: (i, 0))]
    out_shape = [jax.ShapeDtypeStruct((N2, EMB), jnp.float32)]
    if not last:
        out_specs.append(pl.BlockSpec((NUM_BOND, R, EMB), lambda i: (0, i, 0)))
        out_shape.append(jax.ShapeDtypeStruct((NUM_BOND, N2, EMB), jnp.float32))

    res = pl.pallas_call(
        body,
        grid=(NB,),
        in_specs=[
            pl.BlockSpec((R, EMB), lambda i: (i, 0)),
            pl.BlockSpec((2, 8, EMB), lambda i: (0, 0, 0)),
            pl.BlockSpec((1, EMB), lambda i: (0, 0)),
            pl.BlockSpec((1, EMB), lambda i: (0, 0)),
            pl.BlockSpec((NUM_BOND, EMB), lambda i: (0, 0)),
        ],
        out_specs=out_specs,
        out_shape=out_shape,
    )(u, stats, gamma, beta, e_emb)
    if last:
        return res[0], None
    return res[0], res[1]


def _pool(h, batchf):
    def body(b_ref, h_ref, out_ref, acc_ref, cnt_ref):
        i = pl.program_id(0)

        @pl.when(i == 0)
        def _():
            acc_ref[...] = jnp.zeros((2 * B, EMB), jnp.float32)
            cnt_ref[...] = jnp.zeros((2 * B, EMB), jnp.float32)

        g = lax.broadcasted_iota(jnp.int32, (2 * B, R), 0).astype(jnp.float32)
        m = (g == b_ref[0]).astype(jnp.float32)
        acc_ref[...] += jnp.dot(m, h_ref[...], preferred_element_type=jnp.float32)
        cnt_ref[...] += jnp.dot(m, jnp.ones((R, EMB), jnp.float32),
                                preferred_element_type=jnp.float32)

        @pl.when(i == NB - 1)
        def _():
            out_ref[...] = acc_ref[...] / jnp.maximum(cnt_ref[...], 1.0)

    return pl.pallas_call(
        body,
        grid=(NB,),
        in_specs=[
            pl.BlockSpec((1, 1, R), lambda i: (i, 0, 0)),
            pl.BlockSpec((R, EMB), lambda i: (i, 0)),
        ],
        out_specs=pl.BlockSpec((2 * B, EMB), lambda i: (0, 0)),
        out_shape=jax.ShapeDtypeStruct((2 * B, EMB), jnp.float32),
        scratch_shapes=[
            pltpu.VMEM((2 * B, EMB), jnp.float32),
            pltpu.VMEM((2 * B, EMB), jnp.float32),
        ],
    )(batchf, h)


def _head(pooled, context, Wo1, bo1, Wo2, bo2, Wc1, bc1, Wc2, bc2,
          Wm1, bm1, Wm2, bm2, Wm3, bm3):
    def body(p_ref, c_ref, wo1, bo1r, wo2, bo2r, wc1, bc1r, wc2, bc2r,
             wm1, bm1r, wm2, bm2r, wm3, bm3r, o_ref):
        def mm(a, w, bias):
            return jnp.dot(a, w[...], preferred_element_type=jnp.float32) + bias[...]

        pa = p_ref[0:B]
        pb = p_ref[B:2 * B]
        ha = mm(jnp.maximum(mm(pa, wo1, bo1r), 0.0), wo2, bo2r)
        hb = mm(jnp.maximum(mm(pb, wo1, bo1r), 0.0), wo2, bo2r)
        ctx = mm(jnp.maximum(mm(c_ref[...], wc1, bc1r), 0.0), wc2, bc2r)
        z = jnp.concatenate([ha, hb, ctx], axis=1)
        z = jnp.maximum(mm(z, wm1, bm1r), 0.0)
        z = jnp.maximum(mm(z, wm2, bm2r), 0.0)
        o_ref[...] = mm(z, wm3, bm3r)

    args = (pooled, context, Wo1, bo1, Wo2, bo2, Wc1, bc1, Wc2, bc2,
            Wm1, bm1, Wm2, bm2, Wm3, bm3)
    return pl.pallas_call(
        body,
        out_shape=jax.ShapeDtypeStruct((B, 1), jnp.float32),
    )(*args)


def kernel(xA, edge_indexA, edge_attrA, batchA, xB, edge_indexB, edge_attrB,
           batchB, context, params):
    xf = jnp.concatenate([xA[:, 0], xB[:, 0]]).astype(jnp.float32)[:, None]

    pad_g = jnp.arange(K, dtype=jnp.int32)
    pad_d = jnp.full((K,), N, jnp.int32)

    def _prep(ei, ea, node_off):
        d = ei[1].astype(jnp.int32)
        order = jnp.argsort((d >= HN).astype(jnp.int32), stable=True)
        ds = d[order]
        gs = (ea[:, 0].astype(jnp.int32) * N2
              + node_off + ei[0].astype(jnp.int32))[order]
        mid = jnp.sum((d < HN).astype(jnp.int32))
        st = jnp.concatenate([
            jnp.zeros((1,), jnp.int32), mid[None],
            jnp.full((14,), E, jnp.int32),
        ])
        return (jnp.concatenate([gs, pad_g]), jnp.concatenate([ds, pad_d]), st)

    gA, dA, stA = _prep(edge_indexA, edge_attrA, 0)
    gB, dB, stB = _prep(edge_indexB, edge_attrB, N)
    gidx = jnp.concatenate([gA, gB])
    dst = jnp.concatenate([dA, dB])
    starts = jnp.stack([stA, stB])
    zrows = jnp.zeros((PT_LAST + 8, EMB), jnp.float32)
    batchf = jnp.concatenate([batchA, batchB + B]).astype(jnp.float32).reshape(NB, 1, R)

    p = params
    row = lambda v: v[None, :]

    h, T = _embed(xf, p['x_emb'], p['e_emb'])
    for l in range(NUM_LAYER):
        gl = p['gnn'][l]
        aggr = _sc_aggregate(T.reshape(NUM_BOND * N2, EMB), gidx, dst,
                             starts, zrows).reshape(N2, EMB)
        u, stats = _mlp(h, aggr, gl['W1'], row(gl['b1']), gl['W2'], row(gl['b2']))
        h, T = _norm(u, stats, row(gl['gamma']), row(gl['beta']), p['e_emb'],
                     last=(l == NUM_LAYER - 1))

    pooled = _pool(h, batchf)
    return _head(pooled, context,
                 p['Wo1'], row(p['bo1']), p['Wo2'], row(p['bo2']),
                 p['Wc1'], row(p['bc1']), p['Wc2'], row(p['bc2']),
                 p['Wm1'], row(p['bm1']), p['Wm2'], row(p['bm2']),
                 p['Wm3'], row(p['bm3']))


# speedup vs baseline: 4.9048x; 1.3842x over previous
import functools

import jax
import jax.numpy as jnp
from jax import lax
from jax.experimental import pallas as pl
from jax.experimental.pallas import tpu as pltpu
from jax.experimental.pallas import tpu_sc as plsc

N = 10000
E = 160000
B = 64
EMB = 128
NUM_BOND = 6
NUM_ATOM = 120
NUM_LAYER = 5

N2 = 2 * N
E2 = 2 * E
R = 2000
NB = N2 // R
HALF = EMB // 2

SC_SUB = 16
K = 128
PH = 2
HN = N // PH
PT = HN // SC_SUB // 8 * 8
PT_LAST = HN - 15 * PT
ACC_ROWS = HN + 8
EPAD = E + K
NCH = EPAD // K


def _sc_aggregate(T2, edata, starts, zrows):
    mesh = plsc.VectorSubcoreMesh(core_axis_name="c", subcore_axis_name="s")

    @functools.partial(
        pl.kernel,
        out_type=jax.ShapeDtypeStruct((2, N, EMB), jnp.float32),
        mesh=mesh,
        scratch_types=[
            pltpu.VMEM((2, 2, K), jnp.int32),
            pltpu.VMEM((2, K), jnp.int32),
            pltpu.VMEM((2, K, EMB), jnp.float32),
            pltpu.VMEM((2, 16), jnp.int32),
            pltpu.VMEM_SHARED((ACC_ROWS, EMB), jnp.float32),
            pltpu.SemaphoreType.DMA,
            pltpu.SemaphoreType.DMA,
            pltpu.SemaphoreType.DMA,
            pltpu.SemaphoreType.DMA,
            pltpu.SemaphoreType.DMA,
            pltpu.SemaphoreType.DMA,
        ],
    )
    def agg(t_hbm, ed_hbm, st_hbm, z_hbm, out_hbm,
            ed_v, dl_v, msg_v, st_v, acc_sh, sed0, sed1, sg0, sg1, ss0, ss1):
        c = lax.axis_index("c")
        s = lax.axis_index("s")
        sed = (sed0, sed1)
        sg = (sg0, sg1)
        ss = (ss0, ss1)

        pltpu.sync_copy(st_hbm, st_v)
        svec = st_v[c, 0:16]
        trash = HN + (jnp.arange(16, dtype=jnp.int32) & 7)
        row0 = s * PT

        for p in range(PH):
            @pl.when(s < SC_SUB - 1)
            def _():
                pltpu.sync_copy(z_hbm.at[pl.ds(0, PT)],
                                acc_sh.at[pl.ds(row0, PT)])

            @pl.when(s == SC_SUB - 1)
            def _():
                pltpu.sync_copy(z_hbm, acc_sh.at[pl.ds(15 * PT, PT_LAST + 8)])

            plsc.subcore_barrier()

            s0 = svec[p]
            s1 = svec[p + 1]
            clo = s0 // K
            chi = (s1 + K - 1) // K
            nch = chi - clo
            nmy = (nch - s + SC_SUB - 1) // SC_SUB
            cbase = c * NCH + clo + s
            pbase = p * HN

            def chunk(i):
                return ed_hbm.at[cbase + i * SC_SUB]

            def fetch(i, b):
                return pltpu.async_copy(chunk(i), ed_v.at[b], sed[b])

            @pl.when(nmy > 0)
            def _():
                fetch(0, 0)

            @pl.when(nmy > 1)
            def _():
                fetch(1, 1)

            def outer(i, _):
                for b in range(2):
                    tt = 2 * i + b

                    @pl.when(tt < nmy)
                    def _():
                        @pl.when(tt >= 2)
                        def _():
                            pltpu.make_async_copy(
                                msg_v.at[b], acc_sh.at[dl_v.at[b]],
                                ss[b]).wait()

                        pltpu.make_async_copy(chunk(tt), ed_v.at[b],
                                              sed[b]).wait()
                        gcp = pltpu.async_copy(t_hbm.at[ed_v.at[b, 0]],
                                               msg_v.at[b], sg[b])
                        for j in range(K // 16):
                            d = ed_v[b, 1, j * 16:(j + 1) * 16] - pbase
                            ok = (d >= 0) & (d < HN)
                            dl_v[b, j * 16:(j + 1) * 16] = jnp.where(ok, d,
                                                                     trash)
                        gcp.wait()

                        @pl.when(tt + 2 < nmy)
                        def _():
                            fetch(tt + 2, b)

                        pltpu.async_copy(msg_v.at[b],
                                         acc_sh.at[dl_v.at[b]], ss[b],
                                         add=True)
                return 0

            lax.fori_loop(0, (nmy + 1) // 2, outer, 0)

            for b in range(2):
                @pl.when(nmy > b)
                def _():
                    pltpu.make_async_copy(msg_v.at[b],
                                          acc_sh.at[dl_v.at[b]], ss[b]).wait()

            plsc.subcore_barrier()

            @pl.when(s < SC_SUB - 1)
            def _():
                pltpu.sync_copy(acc_sh.at[pl.ds(row0, PT)],
                                out_hbm.at[c].at[pl.ds(pbase + row0, PT)])

            @pl.when(s == SC_SUB - 1)
            def _():
                pltpu.sync_copy(acc_sh.at[pl.ds(15 * PT, PT_LAST)],
                                out_hbm.at[c].at[pl.ds(pbase + 15 * PT,
                                                       PT_LAST)])

            plsc.subcore_barrier()

    return agg(T2, edata, starts, zrows)


def _embed(xf, x_emb, e_emb):
    def body(x_ref, xe_ref, ee_ref, h_ref, t_ref):
        ids = x_ref[...]
        io = lax.broadcasted_iota(jnp.int32, (R, NUM_ATOM), 1).astype(jnp.float32)
        oh = (io == ids).astype(jnp.float32)
        h = jnp.dot(oh, xe_ref[...], preferred_element_type=jnp.float32)
        h_ref[...] = h
        for b in range(NUM_BOND):
            t_ref[b] = jnp.maximum(h + ee_ref[b, :][None, :], 0.0)

    return pl.pallas_call(
        body,
        grid=(NB,),
        in_specs=[
            pl.BlockSpec((R, 1), lambda i: (i, 0)),
            pl.BlockSpec((NUM_ATOM, EMB), lambda i: (0, 0)),
            pl.BlockSpec((NUM_BOND, EMB), lambda i: (0, 0)),
        ],
        out_specs=[
            pl.BlockSpec((R, EMB), lambda i: (i, 0)),
            pl.BlockSpec((NUM_BOND, R, EMB), lambda i: (0, i, 0)),
        ],
        out_shape=[
            jax.ShapeDtypeStruct((N2, EMB), jnp.float32),
            jax.ShapeDtypeStruct((NUM_BOND, N2, EMB), jnp.float32),
        ],
    )(xf, x_emb, e_emb)


def _mlp(h, aggr, W1, b1, W2, b2):
    def body(h_ref, a_ref, w1_ref, b1_ref, w2_ref, b2_ref, u_ref, st_ref):
        i = pl.program_id(0)
        z = h_ref[...] + a_ref[...]
        t = jnp.maximum(jnp.dot(z, w1_ref[...], preferred_element_type=jnp.float32)
                        + b1_ref[...], 0.0)
        u = jnp.dot(t, w2_ref[...], preferred_element_type=jnp.float32) + b2_ref[...]
        u_ref[...] = u

        @pl.when(i % (NB // 2) == 0)
        def _():
            st_ref[...] = jnp.zeros((1, 8, EMB), jnp.float32)

        st_ref[0, 0:1, :] += jnp.sum(u, axis=0, keepdims=True)
        st_ref[0, 1:2, :] += jnp.sum(u * u, axis=0, keepdims=True)

    return pl.pallas_call(
        body,
        grid=(NB,),
        in_specs=[
            pl.BlockSpec((R, EMB), lambda i: (i, 0)),
            pl.BlockSpec((R, EMB), lambda i: (i, 0)),
            pl.BlockSpec((EMB, 2 * EMB), lambda i: (0, 0)),
            pl.BlockSpec((1, 2 * EMB), lambda i: (0, 0)),
            pl.BlockSpec((2 * EMB, EMB), lambda i: (0, 0)),
            pl.BlockSpec((1, EMB), lambda i: (0, 0)),
        ],
        out_specs=[
            pl.BlockSpec((R, EMB), lambda i: (i, 0)),
            pl.BlockSpec((1, 8, EMB), lambda i: (i // (NB // 2), 0, 0)),
        ],
        out_shape=[
            jax.ShapeDtypeStruct((N2, EMB), jnp.float32),
            jax.ShapeDtypeStruct((2, 8, EMB), jnp.float32),
        ],
    )(h, aggr, W1, b1, W2, b2)


def _norm(u, stats, gamma, beta, e_emb, last):
    def body(u_ref, st_ref, g_ref, be_ref, ee_ref, h_ref, *maybe_t):
        i = pl.program_id(0)
        is_a = (i < NB // 2)
        s = jnp.where(is_a, st_ref[0, 0:1, :], st_ref[1, 0:1, :])
        s2 = jnp.where(is_a, st_ref[0, 1:2, :], st_ref[1, 1:2, :])
        mu = s * (1.0 / N)
        var = s2 * (1.0 / N) - mu * mu
        inv = lax.rsqrt(var + 1e-5)
        hn = g_ref[...] * (u_ref[...] - mu) * inv + be_ref[...]
        if not last:
            hn = jnp.maximum(hn, 0.0)
        h_ref[...] = hn
        if not last:
            t_ref = maybe_t[0]
            for b in range(NUM_BOND):
                t_ref[b] = jnp.maximum(hn + ee_ref[b, :][None, :], 0.0)

    out_specs = [pl.BlockSpec((R, EMB), lambda i: (i, 0))]
    out_shape = [jax.ShapeDtypeStruct((N2, EMB), jnp.float32)]
    if not last:
        out_specs.append(pl.BlockSpec((NUM_BOND, R, EMB), lambda i: (0, i, 0)))
        out_shape.append(jax.ShapeDtypeStruct((NUM_BOND, N2, EMB), jnp.float32))

    res = pl.pallas_call(
        body,
        grid=(NB,),
        in_specs=[
            pl.BlockSpec((R, EMB), lambda i: (i, 0)),
            pl.BlockSpec((2, 8, EMB), lambda i: (0, 0, 0)),
            pl.BlockSpec((1, EMB), lambda i: (0, 0)),
            pl.BlockSpec((1, EMB), lambda i: (0, 0)),
            pl.BlockSpec((NUM_BOND, EMB), lambda i: (0, 0)),
        ],
        out_specs=out_specs,
        out_shape=out_shape,
    )(u, stats, gamma, beta, e_emb)
    if last:
        return res[0], None
    return res[0], res[1]


def _pool(h, batchf):
    def body(b_ref, h_ref, out_ref, acc_ref, cnt_ref):
        i = pl.program_id(0)

        @pl.when(i == 0)
        def _():
            acc_ref[...] = jnp.zeros((2 * B, EMB), jnp.float32)
            cnt_ref[...] = jnp.zeros((2 * B, EMB), jnp.float32)

        g = lax.broadcasted_iota(jnp.int32, (2 * B, R), 0).astype(jnp.float32)
        m = (g == b_ref[0]).astype(jnp.float32)
        acc_ref[...] += jnp.dot(m, h_ref[...], preferred_element_type=jnp.float32)
        cnt_ref[...] += jnp.dot(m, jnp.ones((R, EMB), jnp.float32),
                                preferred_element_type=jnp.float32)

        @pl.when(i == NB - 1)
        def _():
            out_ref[...] = acc_ref[...] / jnp.maximum(cnt_ref[...], 1.0)

    return pl.pallas_call(
        body,
        grid=(NB,),
        in_specs=[
            pl.BlockSpec((1, 1, R), lambda i: (i, 0, 0)),
            pl.BlockSpec((R, EMB), lambda i: (i, 0)),
        ],
        out_specs=pl.BlockSpec((2 * B, EMB), lambda i: (0, 0)),
        out_shape=jax.ShapeDtypeStruct((2 * B, EMB), jnp.float32),
        scratch_shapes=[
            pltpu.VMEM((2 * B, EMB), jnp.float32),
            pltpu.VMEM((2 * B, EMB), jnp.float32),
        ],
    )(batchf, h)


def _head(pooled, context, Wo1, bo1, Wo2, bo2, Wc1, bc1, Wc2, bc2,
          Wm1, bm1, Wm2, bm2, Wm3, bm3):
    def body(p_ref, c_ref, wo1, bo1r, wo2, bo2r, wc1, bc1r, wc2, bc2r,
             wm1, bm1r, wm2, bm2r, wm3, bm3r, o_ref):
        def mm(a, w, bias):
            return jnp.dot(a, w[...], preferred_element_type=jnp.float32) + bias[...]

        pa = p_ref[0:B]
        pb = p_ref[B:2 * B]
        ha = mm(jnp.maximum(mm(pa, wo1, bo1r), 0.0), wo2, bo2r)
        hb = mm(jnp.maximum(mm(pb, wo1, bo1r), 0.0), wo2, bo2r)
        ctx = mm(jnp.maximum(mm(c_ref[...], wc1, bc1r), 0.0), wc2, bc2r)
        z = jnp.concatenate([ha, hb, ctx], axis=1)
        z = jnp.maximum(mm(z, wm1, bm1r), 0.0)
        z = jnp.maximum(mm(z, wm2, bm2r), 0.0)
        o_ref[...] = mm(z, wm3, bm3r)

    args = (pooled, context, Wo1, bo1, Wo2, bo2, Wc1, bc1, Wc2, bc2,
            Wm1, bm1, Wm2, bm2, Wm3, bm3)
    return pl.pallas_call(
        body,
        out_shape=jax.ShapeDtypeStruct((B, 1), jnp.float32),
    )(*args)


def kernel(xA, edge_indexA, edge_attrA, batchA, xB, edge_indexB, edge_attrB,
           batchB, context, params):
    xf = jnp.concatenate([xA[:, 0], xB[:, 0]]).astype(jnp.float32)[:, None]

    pad_g = jnp.arange(K, dtype=jnp.int32)
    pad_d = jnp.full((K,), N, jnp.int32)

    def _prep(ei, ea, node_off):
        d = ei[1].astype(jnp.int32)
        order = jnp.argsort((d >= HN).astype(jnp.int32), stable=True)
        ds = d[order]
        gs = (ea[:, 0].astype(jnp.int32) * N2
              + node_off + ei[0].astype(jnp.int32))[order]
        mid = jnp.sum((d < HN).astype(jnp.int32))
        st = jnp.concatenate([
            jnp.zeros((1,), jnp.int32), mid[None],
            jnp.full((14,), E, jnp.int32),
        ])
        ed = jnp.stack([jnp.concatenate([gs, pad_g]).reshape(NCH, K),
                        jnp.concatenate([ds, pad_d]).reshape(NCH, K)], axis=1)
        return ed, st

    edA, stA = _prep(edge_indexA, edge_attrA, 0)
    edB, stB = _prep(edge_indexB, edge_attrB, N)
    edata = jnp.concatenate([edA, edB])
    starts = jnp.stack([stA, stB])
    zrows = jnp.zeros((PT_LAST + 8, EMB), jnp.float32)
    batchf = jnp.concatenate([batchA, batchB + B]).astype(jnp.float32).reshape(NB, 1, R)

    p = params
    row = lambda v: v[None, :]

    h, T = _embed(xf, p['x_emb'], p['e_emb'])
    for l in range(NUM_LAYER):
        gl = p['gnn'][l]
        aggr = _sc_aggregate(T.reshape(NUM_BOND * N2, EMB), edata,
                             starts, zrows).reshape(N2, EMB)
        u, stats = _mlp(h, aggr, gl['W1'], row(gl['b1']), gl['W2'], row(gl['b2']))
        h, T = _norm(u, stats, row(gl['gamma']), row(gl['beta']), p['e_emb'],
                     last=(l == NUM_LAYER - 1))

    pooled = _pool(h, batchf)
    return _head(pooled, context,
                 p['Wo1'], row(p['bo1']), p['Wo2'], row(p['bo2']),
                 p['Wc1'], row(p['bc1']), p['Wc2'], row(p['bc2']),
                 p['Wm1'], row(p['bm1']), p['Wm2'], row(p['bm2']),
                 p['Wm3'], row(p['bm3']))


# speedup vs baseline: 5.0301x; 1.0255x over previous
import functools

import jax
import jax.numpy as jnp
from jax import lax
from jax.experimental import pallas as pl
from jax.experimental.pallas import tpu as pltpu
from jax.experimental.pallas import tpu_sc as plsc

N = 10000
E = 160000
B = 64
EMB = 128
NUM_BOND = 6
NUM_ATOM = 120
NUM_LAYER = 5

N2 = 2 * N
E2 = 2 * E
R = 2000
NB = N2 // R
HALF = EMB // 2

SC_SUB = 16
K = 128
PH = 2
HN = N // PH
PT = HN // SC_SUB // 8 * 8
PT_LAST = HN - 15 * PT
ACC_ROWS = HN + 8
EPAD = E + K
NCH = EPAD // K


def _sc_aggregate(T2, edata, starts, zrows):
    mesh = plsc.VectorSubcoreMesh(core_axis_name="c", subcore_axis_name="s")

    @functools.partial(
        pl.kernel,
        out_type=jax.ShapeDtypeStruct((2, N, EMB), jnp.float32),
        mesh=mesh,
        scratch_types=[
            pltpu.VMEM((2, 2, K), jnp.int32),
            pltpu.VMEM((2, K), jnp.int32),
            pltpu.VMEM((2, K, EMB), jnp.float32),
            pltpu.VMEM((2, 16), jnp.int32),
            pltpu.VMEM_SHARED((ACC_ROWS, EMB), jnp.float32),
            pltpu.SemaphoreType.DMA,
            pltpu.SemaphoreType.DMA,
            pltpu.SemaphoreType.DMA,
            pltpu.SemaphoreType.DMA,
            pltpu.SemaphoreType.DMA,
            pltpu.SemaphoreType.DMA,
        ],
    )
    def agg(t_hbm, ed_hbm, st_hbm, z_hbm, out_hbm,
            ed_v, dl_v, msg_v, st_v, acc_sh, sed0, sed1, sg0, sg1, ss0, ss1):
        c = lax.axis_index("c")
        s = lax.axis_index("s")
        sed = (sed0, sed1)
        sg = (sg0, sg1)
        ss = (ss0, ss1)

        pltpu.sync_copy(st_hbm, st_v)
        svec = st_v[c, 0:16]
        trash = HN + (jnp.arange(16, dtype=jnp.int32) & 7)
        row0 = s * PT

        for p in range(PH):
            @pl.when(s < SC_SUB - 1)
            def _():
                pltpu.sync_copy(z_hbm.at[pl.ds(0, PT)],
                                acc_sh.at[pl.ds(row0, PT)])

            @pl.when(s == SC_SUB - 1)
            def _():
                pltpu.sync_copy(z_hbm, acc_sh.at[pl.ds(15 * PT, PT_LAST + 8)])

            plsc.subcore_barrier()

            s0 = svec[p]
            s1 = svec[p + 1]
            clo = s0 // K
            chi = (s1 + K - 1) // K
            nch = chi - clo
            nmy = (nch - s + SC_SUB - 1) // SC_SUB
            cbase = c * NCH + clo + s
            pbase = p * HN

            def chunk(i):
                return ed_hbm.at[cbase + i * SC_SUB]

            def fetch(i, b):
                return pltpu.async_copy(chunk(i), ed_v.at[b], sed[b])

            @pl.when(nmy > 0)
            def _():
                fetch(0, 0)

            @pl.when(nmy > 1)
            def _():
                fetch(1, 1)

            def outer(i, _):
                for b in range(2):
                    tt = 2 * i + b

                    @pl.when(tt < nmy)
                    def _():
                        @pl.when(tt >= 2)
                        def _():
                            pltpu.make_async_copy(
                                msg_v.at[b], acc_sh.at[dl_v.at[b]],
                                ss[b]).wait()

                        pltpu.make_async_copy(chunk(tt), ed_v.at[b],
                                              sed[b]).wait()
                        gcp = pltpu.async_copy(t_hbm.at[ed_v.at[b, 0]],
                                               msg_v.at[b], sg[b])
                        for j in range(K // 16):
                            d = ed_v[b, 1, j * 16:(j + 1) * 16] - pbase
                            ok = (d >= 0) & (d < HN)
                            dl_v[b, j * 16:(j + 1) * 16] = jnp.where(ok, d,
                                                                     trash)
                        gcp.wait()

                        @pl.when(tt + 2 < nmy)
                        def _():
                            fetch(tt + 2, b)

                        pltpu.async_copy(msg_v.at[b],
                                         acc_sh.at[dl_v.at[b]], ss[b],
                                         add=True)
                return 0

            lax.fori_loop(0, (nmy + 1) // 2, outer, 0)

            for b in range(2):
                @pl.when(nmy > b)
                def _():
                    pltpu.make_async_copy(msg_v.at[b],
                                          acc_sh.at[dl_v.at[b]], ss[b]).wait()

            plsc.subcore_barrier()

            @pl.when(s < SC_SUB - 1)
            def _():
                pltpu.sync_copy(acc_sh.at[pl.ds(row0, PT)],
                                out_hbm.at[c].at[pl.ds(pbase + row0, PT)])

            @pl.when(s == SC_SUB - 1)
            def _():
                pltpu.sync_copy(acc_sh.at[pl.ds(15 * PT, PT_LAST)],
                                out_hbm.at[c].at[pl.ds(pbase + 15 * PT,
                                                       PT_LAST)])

            plsc.subcore_barrier()

    return agg(T2, edata, starts, zrows)


def _embed(xf, x_emb, e_emb):
    def body(x_ref, xe_ref, ee_ref, h_ref, t_ref):
        ids = x_ref[...]
        io = lax.broadcasted_iota(jnp.int32, (R, NUM_ATOM), 1).astype(jnp.float32)
        oh = (io == ids).astype(jnp.float32)
        h = jnp.dot(oh, xe_ref[...], preferred_element_type=jnp.float32)
        h_ref[...] = h
        for b in range(NUM_BOND):
            t_ref[b] = jnp.maximum(h + ee_ref[b, :][None, :], 0.0)

    return pl.pallas_call(
        body,
        grid=(NB,),
        in_specs=[
            pl.BlockSpec((R, 1), lambda i: (i, 0)),
            pl.BlockSpec((NUM_ATOM, EMB), lambda i: (0, 0)),
            pl.BlockSpec((NUM_BOND, EMB), lambda i: (0, 0)),
        ],
        out_specs=[
            pl.BlockSpec((R, EMB), lambda i: (i, 0)),
            pl.BlockSpec((NUM_BOND, R, EMB), lambda i: (0, i, 0)),
        ],
        out_shape=[
            jax.ShapeDtypeStruct((N2, EMB), jnp.float32),
            jax.ShapeDtypeStruct((NUM_BOND, N2, EMB), jnp.float32),
        ],
    )(xf, x_emb, e_emb)


def _mlp_norm(h, aggr, W1, b1, W2, b2, gamma, beta, e_emb, last):
    def body(h_ref, a_ref, w1_ref, b1_ref, w2_ref, b2_ref, g_ref, be_ref,
             ee_ref, h_out, *rest):
        if last:
            (u_buf, st_buf) = rest
            t_out = None
        else:
            (t_out, u_buf, st_buf) = rest
        i = pl.program_id(0)

        @pl.when(i == 0)
        def _():
            st_buf[...] = jnp.zeros((2, 8, EMB), jnp.float32)

        @pl.when(i < NB)
        def _():
            z = h_ref[...] + a_ref[...]
            t = jnp.maximum(
                jnp.dot(z, w1_ref[...], preferred_element_type=jnp.float32)
                + b1_ref[...], 0.0)
            u = jnp.dot(t, w2_ref[...],
                        preferred_element_type=jnp.float32) + b2_ref[...]
            u_buf[pl.ds(i * R, R), :] = u
            su = jnp.sum(u, axis=0, keepdims=True)
            s2 = jnp.sum(u * u, axis=0, keepdims=True)

            @pl.when(i < NB // 2)
            def _():
                st_buf[0, 0:1, :] += su
                st_buf[0, 1:2, :] += s2

            @pl.when(i >= NB // 2)
            def _():
                st_buf[1, 0:1, :] += su
                st_buf[1, 1:2, :] += s2

        @pl.when(i >= NB)
        def _():
            j = i - NB
            is_a = (j < NB // 2)
            sm = jnp.where(is_a, st_buf[0, 0:1, :], st_buf[1, 0:1, :])
            s2m = jnp.where(is_a, st_buf[0, 1:2, :], st_buf[1, 1:2, :])
            mu = sm * (1.0 / N)
            var = s2m * (1.0 / N) - mu * mu
            inv = lax.rsqrt(var + 1e-5)
            u = u_buf[pl.ds(j * R, R), :]
            hn = g_ref[...] * (u - mu) * inv + be_ref[...]
            if not last:
                hn = jnp.maximum(hn, 0.0)
            h_out[...] = hn
            if not last:
                for b in range(NUM_BOND):
                    t_out[b] = jnp.maximum(hn + ee_ref[b, :][None, :], 0.0)

    def blk(i):
        return jnp.where(i < NB, i, 0)

    def blk_out(i):
        return jnp.where(i >= NB, i - NB, 0)

    out_specs = [pl.BlockSpec((R, EMB), lambda i: (blk_out(i), 0))]
    out_shape = [jax.ShapeDtypeStruct((N2, EMB), jnp.float32)]
    if not last:
        out_specs.append(
            pl.BlockSpec((NUM_BOND, R, EMB), lambda i: (0, blk_out(i), 0)))
        out_shape.append(
            jax.ShapeDtypeStruct((NUM_BOND, N2, EMB), jnp.float32))

    res = pl.pallas_call(
        body,
        grid=(2 * NB,),
        in_specs=[
            pl.BlockSpec((R, EMB), lambda i: (blk(i), 0)),
            pl.BlockSpec((R, EMB), lambda i: (blk(i), 0)),
            pl.BlockSpec((EMB, 2 * EMB), lambda i: (0, 0)),
            pl.BlockSpec((1, 2 * EMB), lambda i: (0, 0)),
            pl.BlockSpec((2 * EMB, EMB), lambda i: (0, 0)),
            pl.BlockSpec((1, EMB), lambda i: (0, 0)),
            pl.BlockSpec((1, EMB), lambda i: (0, 0)),
            pl.BlockSpec((1, EMB), lambda i: (0, 0)),
            pl.BlockSpec((NUM_BOND, EMB), lambda i: (0, 0)),
        ],
        out_specs=out_specs,
        out_shape=out_shape,
        scratch_shapes=[
            pltpu.VMEM((N2, EMB), jnp.float32),
            pltpu.VMEM((2, 8, EMB), jnp.float32),
        ],
    )(h, aggr, W1, b1, W2, b2, gamma, beta, e_emb)
    if last:
        return res[0], None
    return res[0], res[1]


def _pool(h, batchf):
    def body(b_ref, h_ref, out_ref, acc_ref, cnt_ref):
        i = pl.program_id(0)

        @pl.when(i == 0)
        def _():
            acc_ref[...] = jnp.zeros((2 * B, EMB), jnp.float32)
            cnt_ref[...] = jnp.zeros((2 * B, EMB), jnp.float32)

        g = lax.broadcasted_iota(jnp.int32, (2 * B, R), 0).astype(jnp.float32)
        m = (g == b_ref[0]).astype(jnp.float32)
        acc_ref[...] += jnp.dot(m, h_ref[...], preferred_element_type=jnp.float32)
        cnt_ref[...] += jnp.dot(m, jnp.ones((R, EMB), jnp.float32),
                                preferred_element_type=jnp.float32)

        @pl.when(i == NB - 1)
        def _():
            out_ref[...] = acc_ref[...] / jnp.maximum(cnt_ref[...], 1.0)

    return pl.pallas_call(
        body,
        grid=(NB,),
        in_specs=[
            pl.BlockSpec((1, 1, R), lambda i: (i, 0, 0)),
            pl.BlockSpec((R, EMB), lambda i: (i, 0)),
        ],
        out_specs=pl.BlockSpec((2 * B, EMB), lambda i: (0, 0)),
        out_shape=jax.ShapeDtypeStruct((2 * B, EMB), jnp.float32),
        scratch_shapes=[
            pltpu.VMEM((2 * B, EMB), jnp.float32),
            pltpu.VMEM((2 * B, EMB), jnp.float32),
        ],
    )(batchf, h)


def _head(pooled, context, Wo1, bo1, Wo2, bo2, Wc1, bc1, Wc2, bc2,
          Wm1, bm1, Wm2, bm2, Wm3, bm3):
    def body(p_ref, c_ref, wo1, bo1r, wo2, bo2r, wc1, bc1r, wc2, bc2r,
             wm1, bm1r, wm2, bm2r, wm3, bm3r, o_ref):
        def mm(a, w, bias):
            return jnp.dot(a, w[...], preferred_element_type=jnp.float32) + bias[...]

        pa = p_ref[0:B]
        pb = p_ref[B:2 * B]
        ha = mm(jnp.maximum(mm(pa, wo1, bo1r), 0.0), wo2, bo2r)
        hb = mm(jnp.maximum(mm(pb, wo1, bo1r), 0.0), wo2, bo2r)
        ctx = mm(jnp.maximum(mm(c_ref[...], wc1, bc1r), 0.0), wc2, bc2r)
        z = jnp.concatenate([ha, hb, ctx], axis=1)
        z = jnp.maximum(mm(z, wm1, bm1r), 0.0)
        z = jnp.maximum(mm(z, wm2, bm2r), 0.0)
        o_ref[...] = mm(z, wm3, bm3r)

    args = (pooled, context, Wo1, bo1, Wo2, bo2, Wc1, bc1, Wc2, bc2,
            Wm1, bm1, Wm2, bm2, Wm3, bm3)
    return pl.pallas_call(
        body,
        out_shape=jax.ShapeDtypeStruct((B, 1), jnp.float32),
    )(*args)


def kernel(xA, edge_indexA, edge_attrA, batchA, xB, edge_indexB, edge_attrB,
           batchB, context, params):
    xf = jnp.concatenate([xA[:, 0], xB[:, 0]]).astype(jnp.float32)[:, None]

    pad_g = jnp.arange(K, dtype=jnp.int32)
    pad_d = jnp.full((K,), N, jnp.int32)

    def _prep(ei, ea, node_off):
        d = ei[1].astype(jnp.int32)
        order = jnp.argsort((d >= HN).astype(jnp.int32), stable=True)
        ds = d[order]
        gs = (ea[:, 0].astype(jnp.int32) * N2
              + node_off + ei[0].astype(jnp.int32))[order]
        mid = jnp.sum((d < HN).astype(jnp.int32))
        st = jnp.concatenate([
            jnp.zeros((1,), jnp.int32), mid[None],
            jnp.full((14,), E, jnp.int32),
        ])
        ed = jnp.stack([jnp.concatenate([gs, pad_g]).reshape(NCH, K),
                        jnp.concatenate([ds, pad_d]).reshape(NCH, K)], axis=1)
        return ed, st

    edA, stA = _prep(edge_indexA, edge_attrA, 0)
    edB, stB = _prep(edge_indexB, edge_attrB, N)
    edata = jnp.concatenate([edA, edB])
    starts = jnp.stack([stA, stB])
    zrows = jnp.zeros((PT_LAST + 8, EMB), jnp.float32)
    batchf = jnp.concatenate([batchA, batchB + B]).astype(jnp.float32).reshape(NB, 1, R)

    p = params
    row = lambda v: v[None, :]

    h, T = _embed(xf, p['x_emb'], p['e_emb'])
    for l in range(NUM_LAYER):
        gl = p['gnn'][l]
        aggr = _sc_aggregate(T.reshape(NUM_BOND * N2, EMB), edata,
                             starts, zrows).reshape(N2, EMB)
        h, T = _mlp_norm(h, aggr, gl['W1'], row(gl['b1']), gl['W2'],
                         row(gl['b2']), row(gl['gamma']), row(gl['beta']),
                         p['e_emb'], last=(l == NUM_LAYER - 1))

    pooled = _pool(h, batchf)
    return _head(pooled, context,
                 p['Wo1'], row(p['bo1']), p['Wo2'], row(p['bo2']),
                 p['Wc1'], row(p['bc1']), p['Wc2'], row(p['bc2']),
                 p['Wm1'], row(p['bm1']), p['Wm2'], row(p['bm2']),
                 p['Wm3'], row(p['bm3']))


# speedup vs baseline: 5.3455x; 1.0627x over previous
import functools

import jax
import jax.numpy as jnp
from jax import lax
from jax.experimental import pallas as pl
from jax.experimental.pallas import tpu as pltpu
from jax.experimental.pallas import tpu_sc as plsc

N = 10000
E = 160000
B = 64
EMB = 128
NUM_BOND = 6
NUM_ATOM = 120
NUM_LAYER = 5

R = 2000
NB = N // R

SC_SUB = 16
K = 128
HN = N // 2
PT = HN // SC_SUB // 8 * 8
PT_LAST = HN - 15 * PT
ACC_ROWS = HN + 8
EPAD = E + K
NCH = EPAD // K


def _sc_aggregate(T2, edata, starts, zrows):
    mesh = plsc.VectorSubcoreMesh(core_axis_name="c", subcore_axis_name="s")

    @functools.partial(
        pl.kernel,
        out_type=jax.ShapeDtypeStruct((2, HN, EMB), jnp.float32),
        mesh=mesh,
        scratch_types=[
            pltpu.VMEM((2, 2, K), jnp.int32),
            pltpu.VMEM((2, K), jnp.int32),
            pltpu.VMEM((2, K, EMB), jnp.float32),
            pltpu.VMEM((24,), jnp.int32),
            pltpu.VMEM_SHARED((ACC_ROWS, EMB), jnp.float32),
            pltpu.SemaphoreType.DMA,
            pltpu.SemaphoreType.DMA,
            pltpu.SemaphoreType.DMA,
            pltpu.SemaphoreType.DMA,
            pltpu.SemaphoreType.DMA,
            pltpu.SemaphoreType.DMA,
        ],
    )
    def agg(t_hbm, ed_hbm, st_hbm, z_hbm, out_hbm,
            ed_v, dl_v, msg_v, st_v, acc_sh, sed0, sed1, sg0, sg1, ss0, ss1):
        c = lax.axis_index("c")
        s = lax.axis_index("s")
        sed = (sed0, sed1)
        sg = (sg0, sg1)
        ss = (ss0, ss1)

        pltpu.sync_copy(st_hbm, st_v)
        svec = st_v[pl.ds(c, 16)]
        trash = HN + (jnp.arange(16, dtype=jnp.int32) & 7)
        row0 = s * PT

        @pl.when(s < SC_SUB - 1)
        def _():
            pltpu.sync_copy(z_hbm.at[pl.ds(0, PT)], acc_sh.at[pl.ds(row0, PT)])

        @pl.when(s == SC_SUB - 1)
        def _():
            pltpu.sync_copy(z_hbm, acc_sh.at[pl.ds(15 * PT, PT_LAST + 8)])

        plsc.subcore_barrier()

        s0 = svec[0]
        s1 = svec[1]
        clo = s0 // K
        chi = (s1 + K - 1) // K
        nch = chi - clo
        nmy = (nch - s + SC_SUB - 1) // SC_SUB
        cbase = clo + s
        pbase = c * HN

        def chunk(i):
            return ed_hbm.at[cbase + i * SC_SUB]

        def fetch(i, b):
            return pltpu.async_copy(chunk(i), ed_v.at[b], sed[b])

        @pl.when(nmy > 0)
        def _():
            fetch(0, 0)

        @pl.when(nmy > 1)
        def _():
            fetch(1, 1)

        def outer(i, _):
            for b in range(2):
                tt = 2 * i + b

                @pl.when(tt < nmy)
                def _():
                    @pl.when(tt >= 2)
                    def _():
                        pltpu.make_async_copy(
                            msg_v.at[b], acc_sh.at[dl_v.at[b]], ss[b]).wait()

                    pltpu.make_async_copy(chunk(tt), ed_v.at[b], sed[b]).wait()
                    gcp = pltpu.async_copy(t_hbm.at[ed_v.at[b, 0]],
                                           msg_v.at[b], sg[b])
                    for j in range(K // 16):
                        d = ed_v[b, 1, j * 16:(j + 1) * 16] - pbase
                        ok = (d >= 0) & (d < HN)
                        dl_v[b, j * 16:(j + 1) * 16] = jnp.where(ok, d, trash)
                    gcp.wait()

                    @pl.when(tt + 2 < nmy)
                    def _():
                        fetch(tt + 2, b)

                    pltpu.async_copy(msg_v.at[b], acc_sh.at[dl_v.at[b]],
                                     ss[b], add=True)
            return 0

        lax.fori_loop(0, (nmy + 1) // 2, outer, 0)

        for b in range(2):
            @pl.when(nmy > b)
            def _():
                pltpu.make_async_copy(msg_v.at[b], acc_sh.at[dl_v.at[b]],
                                      ss[b]).wait()

        plsc.subcore_barrier()

        @pl.when(s < SC_SUB - 1)
        def _():
            pltpu.sync_copy(acc_sh.at[pl.ds(row0, PT)],
                            out_hbm.at[c].at[pl.ds(row0, PT)])

        @pl.when(s == SC_SUB - 1)
        def _():
            pltpu.sync_copy(acc_sh.at[pl.ds(15 * PT, PT_LAST)],
                            out_hbm.at[c].at[pl.ds(15 * PT, PT_LAST)])

    return agg(T2, edata, starts, zrows)


def _embed(xf, x_emb, e_emb):
    def body(x_ref, xe_ref, ee_ref, h_ref, t_ref):
        ids = x_ref[...]
        io = lax.broadcasted_iota(jnp.int32, (R, NUM_ATOM), 1).astype(jnp.float32)
        oh = (io == ids).astype(jnp.float32)
        h = jnp.dot(oh, xe_ref[...], preferred_element_type=jnp.float32)
        h_ref[...] = h
        for b in range(NUM_BOND):
            t_ref[b] = jnp.maximum(h + ee_ref[b, :][None, :], 0.0)

    return pl.pallas_call(
        body,
        grid=(NB,),
        in_specs=[
            pl.BlockSpec((R, 1), lambda i: (i, 0)),
            pl.BlockSpec((NUM_ATOM, EMB), lambda i: (0, 0)),
            pl.BlockSpec((NUM_BOND, EMB), lambda i: (0, 0)),
        ],
        out_specs=[
            pl.BlockSpec((R, EMB), lambda i: (i, 0)),
            pl.BlockSpec((NUM_BOND, R, EMB), lambda i: (0, i, 0)),
        ],
        out_shape=[
            jax.ShapeDtypeStruct((N, EMB), jnp.float32),
            jax.ShapeDtypeStruct((NUM_BOND, N, EMB), jnp.float32),
        ],
    )(xf, x_emb, e_emb)


def _mlp_norm(h, aggr, W1, b1, W2, b2, gamma, beta, e_emb, last):
    def body(h_ref, a_ref, w1_ref, b1_ref, w2_ref, b2_ref, g_ref, be_ref,
             ee_ref, h_out, *rest):
        if last:
            (u_buf, st_buf) = rest
            t_out = None
        else:
            (t_out, u_buf, st_buf) = rest
        i = pl.program_id(0)

        @pl.when(i == 0)
        def _():
            st_buf[...] = jnp.zeros((8, EMB), jnp.float32)

        @pl.when(i < NB)
        def _():
            z = h_ref[...] + a_ref[...]
            t = jnp.maximum(
                jnp.dot(z, w1_ref[...], preferred_element_type=jnp.float32)
                + b1_ref[...], 0.0)
            u = jnp.dot(t, w2_ref[...],
                        preferred_element_type=jnp.float32) + b2_ref[...]
            u_buf[pl.ds(i * R, R), :] = u
            st_buf[0:1, :] += jnp.sum(u, axis=0, keepdims=True)
            st_buf[1:2, :] += jnp.sum(u * u, axis=0, keepdims=True)

        @pl.when(i >= NB)
        def _():
            j = i - NB
            mu = st_buf[0:1, :] * (1.0 / N)
            var = st_buf[1:2, :] * (1.0 / N) - mu * mu
            inv = lax.rsqrt(var + 1e-5)
            u = u_buf[pl.ds(j * R, R), :]
            hn = g_ref[...] * (u - mu) * inv + be_ref[...]
            if not last:
                hn = jnp.maximum(hn, 0.0)
            h_out[...] = hn
            if not last:
                for b in range(NUM_BOND):
                    t_out[b] = jnp.maximum(hn + ee_ref[b, :][None, :], 0.0)

    def blk(i):
        return jnp.where(i < NB, i, 0)

    def blk_out(i):
        return jnp.where(i >= NB, i - NB, 0)

    out_specs = [pl.BlockSpec((R, EMB), lambda i: (blk_out(i), 0))]
    out_shape = [jax.ShapeDtypeStruct((N, EMB), jnp.float32)]
    if not last:
        out_specs.append(
            pl.BlockSpec((NUM_BOND, R, EMB), lambda i: (0, blk_out(i), 0)))
        out_shape.append(
            jax.ShapeDtypeStruct((NUM_BOND, N, EMB), jnp.float32))

    res = pl.pallas_call(
        body,
        grid=(2 * NB,),
        in_specs=[
            pl.BlockSpec((R, EMB), lambda i: (blk(i), 0)),
            pl.BlockSpec((R, EMB), lambda i: (blk(i), 0)),
            pl.BlockSpec((EMB, 2 * EMB), lambda i: (0, 0)),
            pl.BlockSpec((1, 2 * EMB), lambda i: (0, 0)),
            pl.BlockSpec((2 * EMB, EMB), lambda i: (0, 0)),
            pl.BlockSpec((1, EMB), lambda i: (0, 0)),
            pl.BlockSpec((1, EMB), lambda i: (0, 0)),
            pl.BlockSpec((1, EMB), lambda i: (0, 0)),
            pl.BlockSpec((NUM_BOND, EMB), lambda i: (0, 0)),
        ],
        out_specs=out_specs,
        out_shape=out_shape,
        scratch_shapes=[
            pltpu.VMEM((N, EMB), jnp.float32),
            pltpu.VMEM((8, EMB), jnp.float32),
        ],
    )(h, aggr, W1, b1, W2, b2, gamma, beta, e_emb)
    if last:
        return res[0], None
    return res[0], res[1]


def _pool(h, batchf):
    def body(b_ref, h_ref, out_ref, acc_ref, cnt_ref):
        i = pl.program_id(0)

        @pl.when(i == 0)
        def _():
            acc_ref[...] = jnp.zeros((B, EMB), jnp.float32)
            cnt_ref[...] = jnp.zeros((B, EMB), jnp.float32)

        g = lax.broadcasted_iota(jnp.int32, (B, R), 0).astype(jnp.float32)
        m = (g == b_ref[0]).astype(jnp.float32)
        acc_ref[...] += jnp.dot(m, h_ref[...], preferred_element_type=jnp.float32)
        cnt_ref[...] += jnp.dot(m, jnp.ones((R, EMB), jnp.float32),
                                preferred_element_type=jnp.float32)

        @pl.when(i == NB - 1)
        def _():
            out_ref[...] = acc_ref[...] / jnp.maximum(cnt_ref[...], 1.0)

    return pl.pallas_call(
        body,
        grid=(NB,),
        in_specs=[
            pl.BlockSpec((1, 1, R), lambda i: (i, 0, 0)),
            pl.BlockSpec((R, EMB), lambda i: (i, 0)),
        ],
        out_specs=pl.BlockSpec((B, EMB), lambda i: (0, 0)),
        out_shape=jax.ShapeDtypeStruct((B, EMB), jnp.float32),
        scratch_shapes=[
            pltpu.VMEM((B, EMB), jnp.float32),
            pltpu.VMEM((B, EMB), jnp.float32),
        ],
    )(batchf, h)


def _head(pa, pb, context, Wo1, bo1, Wo2, bo2, Wc1, bc1, Wc2, bc2,
          Wm1, bm1, Wm2, bm2, Wm3, bm3):
    def body(pa_ref, pb_ref, c_ref, wo1, bo1r, wo2, bo2r, wc1, bc1r, wc2,
             bc2r, wm1, bm1r, wm2, bm2r, wm3, bm3r, o_ref):
        def mm(a, w, bias):
            return jnp.dot(a, w[...], preferred_element_type=jnp.float32) + bias[...]

        ha = mm(jnp.maximum(mm(pa_ref[...], wo1, bo1r), 0.0), wo2, bo2r)
        hb = mm(jnp.maximum(mm(pb_ref[...], wo1, bo1r), 0.0), wo2, bo2r)
        ctx = mm(jnp.maximum(mm(c_ref[...], wc1, bc1r), 0.0), wc2, bc2r)
        z = jnp.concatenate([ha, hb, ctx], axis=1)
        z = jnp.maximum(mm(z, wm1, bm1r), 0.0)
        z = jnp.maximum(mm(z, wm2, bm2r), 0.0)
        o_ref[...] = mm(z, wm3, bm3r)

    args = (pa, pb, context, Wo1, bo1, Wo2, bo2, Wc1, bc1, Wc2, bc2,
            Wm1, bm1, Wm2, bm2, Wm3, bm3)
    return pl.pallas_call(
        body,
        out_shape=jax.ShapeDtypeStruct((B, 1), jnp.float32),
    )(*args)


def kernel(xA, edge_indexA, edge_attrA, batchA, xB, edge_indexB, edge_attrB,
           batchB, context, params):
    pad_g = jnp.arange(K, dtype=jnp.int32)
    pad_d = jnp.full((K,), N, jnp.int32)

    def _prep(ei, ea):
        d = ei[1].astype(jnp.int32)
        order = jnp.argsort((d >= HN).astype(jnp.int32), stable=True)
        ds = d[order]
        gs = (ea[:, 0].astype(jnp.int32) * N + ei[0].astype(jnp.int32))[order]
        mid = jnp.sum((d < HN).astype(jnp.int32))
        st = jnp.concatenate([
            jnp.zeros((1,), jnp.int32), mid[None],
            jnp.full((22,), E, jnp.int32),
        ])
        ed = jnp.stack([jnp.concatenate([gs, pad_g]).reshape(NCH, K),
                        jnp.concatenate([ds, pad_d]).reshape(NCH, K)], axis=1)
        return ed, st

    edA, stA = _prep(edge_indexA, edge_attrA)
    edB, stB = _prep(edge_indexB, edge_attrB)
    zrows = jnp.zeros((PT_LAST + 8, EMB), jnp.float32)

    xfA = xA[:, 0].astype(jnp.float32)[:, None]
    xfB = xB[:, 0].astype(jnp.float32)[:, None]
    bfA = batchA.astype(jnp.float32).reshape(NB, 1, R)
    bfB = batchB.astype(jnp.float32).reshape(NB, 1, R)

    p = params
    row = lambda v: v[None, :]

    hA, TA = _embed(xfA, p['x_emb'], p['e_emb'])
    hB, TB = _embed(xfB, p['x_emb'], p['e_emb'])
    for l in range(NUM_LAYER):
        gl = p['gnn'][l]
        aggrA = _sc_aggregate(TA.reshape(NUM_BOND * N, EMB), edA, stA,
                              zrows).reshape(N, EMB)
        aggrB = _sc_aggregate(TB.reshape(NUM_BOND * N, EMB), edB, stB,
                              zrows).reshape(N, EMB)
        args = (gl['W1'], row(gl['b1']), gl['W2'], row(gl['b2']),
                row(gl['gamma']), row(gl['beta']), p['e_emb'])
        hA, TA = _mlp_norm(hA, aggrA, *args, last=(l == NUM_LAYER - 1))
        hB, TB = _mlp_norm(hB, aggrB, *args, last=(l == NUM_LAYER - 1))

    pa = _pool(hA, bfA)
    pb = _pool(hB, bfB)
    return _head(pa, pb, context,
                 p['Wo1'], row(p['bo1']), p['Wo2'], row(p['bo2']),
                 p['Wc1'], row(p['bc1']), p['Wc2'], row(p['bc2']),
                 p['Wm1'], row(p['bm1']), p['Wm2'], row(p['bm2']),
                 p['Wm3'], row(p['bm3']))


# speedup vs baseline: 5.9954x; 1.1216x over previous
import functools

import jax
import jax.numpy as jnp
from jax import lax
from jax.experimental import pallas as pl
from jax.experimental.pallas import tpu as pltpu
from jax.experimental.pallas import tpu_sc as plsc

N = 10000
E = 160000
B = 64
EMB = 128
NUM_BOND = 6
NUM_ATOM = 120
NUM_LAYER = 5

R = 2000
NB = N // R

SC_SUB = 16
K = 128
HN = N // 2
PT = HN // SC_SUB // 8 * 8
PT_LAST = HN - 15 * PT
ACC_ROWS = HN + 8
EPAD = E + K
NCH = EPAD // K


def _sc_aggregate(T2, edata, starts, zrows):
    mesh = plsc.VectorSubcoreMesh(core_axis_name="c", subcore_axis_name="s")

    @functools.partial(
        pl.kernel,
        out_type=jax.ShapeDtypeStruct((2, HN, EMB), jnp.float32),
        mesh=mesh,
        scratch_types=[
            pltpu.VMEM((3, 2, K), jnp.int32),
            pltpu.VMEM((3, K), jnp.int32),
            pltpu.VMEM((3, K, EMB), jnp.float32),
            pltpu.VMEM((24,), jnp.int32),
            pltpu.VMEM_SHARED((ACC_ROWS, EMB), jnp.float32),
            pltpu.SemaphoreType.DMA,
            pltpu.SemaphoreType.DMA,
            pltpu.SemaphoreType.DMA,
            pltpu.SemaphoreType.DMA,
            pltpu.SemaphoreType.DMA,
            pltpu.SemaphoreType.DMA,
            pltpu.SemaphoreType.DMA,
            pltpu.SemaphoreType.DMA,
            pltpu.SemaphoreType.DMA,
        ],
    )
    def agg(t_hbm, ed_hbm, st_hbm, z_hbm, out_hbm,
            ed_v, dl_v, msg_v, st_v, acc_sh,
            sed0, sed1, sed2, sg0, sg1, sg2, ss0, ss1, ss2):
        c = lax.axis_index("c")
        s = lax.axis_index("s")
        sed = (sed0, sed1, sed2)
        sg = (sg0, sg1, sg2)
        ss = (ss0, ss1, ss2)

        pltpu.sync_copy(st_hbm, st_v)
        svec = st_v[pl.ds(c, 16)]
        trash = HN + (jnp.arange(16, dtype=jnp.int32) & 7)
        row0 = s * PT

        @pl.when(s < SC_SUB - 1)
        def _():
            pltpu.sync_copy(z_hbm.at[pl.ds(0, PT)], acc_sh.at[pl.ds(row0, PT)])

        @pl.when(s == SC_SUB - 1)
        def _():
            pltpu.sync_copy(z_hbm, acc_sh.at[pl.ds(15 * PT, PT_LAST + 8)])

        plsc.subcore_barrier()

        s0 = svec[0]
        s1 = svec[1]
        clo = s0 // K
        chi = (s1 + K - 1) // K
        nch = chi - clo
        nmy = (nch - s + SC_SUB - 1) // SC_SUB
        cbase = clo + s
        pbase = c * HN

        def chunk(i):
            return ed_hbm.at[cbase + i * SC_SUB]

        def fetch(i, b):
            return pltpu.async_copy(chunk(i), ed_v.at[b], sed[b])

        def remap(b):
            for j in range(K // 16):
                d = ed_v[b, 1, j * 16:(j + 1) * 16] - pbase
                ok = (d >= 0) & (d < HN)
                dl_v[b, j * 16:(j + 1) * 16] = jnp.where(ok, d, trash)

        def gather(i, b):
            return pltpu.async_copy(t_hbm.at[ed_v.at[b, 0]], msg_v.at[b],
                                    sg[b])

        def wait_scatter(b):
            pltpu.make_async_copy(msg_v.at[b], acc_sh.at[dl_v.at[b]],
                                  ss[b]).wait()

        @pl.when(nmy > 0)
        def _():
            fetch(0, 0)
            pltpu.make_async_copy(chunk(0), ed_v.at[0], sed[0]).wait()
            gather(0, 0)
            remap(0)

        @pl.when(nmy > 1)
        def _():
            fetch(1, 1)

        def outer(i, _):
            for q in range(3):
                tt = 3 * i + q
                b = q
                bn = (q + 1) % 3
                b2 = (q + 2) % 3

                @pl.when(tt < nmy)
                def _():
                    @pl.when(tt + 1 < nmy)
                    def _():
                        pltpu.make_async_copy(chunk(tt + 1), ed_v.at[bn],
                                              sed[bn]).wait()

                        @pl.when(tt >= 2)
                        def _():
                            wait_scatter(bn)

                        gather(tt + 1, bn)
                        remap(bn)

                        @pl.when(tt + 2 < nmy)
                        def _():
                            fetch(tt + 2, b2)

                    pltpu.make_async_copy(t_hbm.at[ed_v.at[b, 0]],
                                          msg_v.at[b], sg[b]).wait()
                    pltpu.async_copy(msg_v.at[b], acc_sh.at[dl_v.at[b]],
                                     ss[b], add=True)
            return 0

        lax.fori_loop(0, (nmy + 2) // 3, outer, 0)

        for b in range(3):
            @pl.when((((nmy - 1) % 3 == b) & (nmy >= 1))
                     | (((nmy - 2) % 3 == b) & (nmy >= 2))
                     | (((nmy - 3) % 3 == b) & (nmy >= 3)))
            def _():
                wait_scatter(b)

        plsc.subcore_barrier()

        @pl.when(s < SC_SUB - 1)
        def _():
            pltpu.sync_copy(acc_sh.at[pl.ds(row0, PT)],
                            out_hbm.at[c].at[pl.ds(row0, PT)])

        @pl.when(s == SC_SUB - 1)
        def _():
            pltpu.sync_copy(acc_sh.at[pl.ds(15 * PT, PT_LAST)],
                            out_hbm.at[c].at[pl.ds(15 * PT, PT_LAST)])

    return agg(T2, edata, starts, zrows)


def _embed(xf, x_emb, e_emb):
    def body(x_ref, xe_ref, ee_ref, h_ref, t_ref):
        ids = x_ref[...]
        io = lax.broadcasted_iota(jnp.int32, (R, NUM_ATOM), 1).astype(jnp.float32)
        oh = (io == ids).astype(jnp.float32)
        h = jnp.dot(oh, xe_ref[...], preferred_element_type=jnp.float32)
        h_ref[...] = h
        for b in range(NUM_BOND):
            t_ref[b] = jnp.maximum(h + ee_ref[b, :][None, :], 0.0)

    return pl.pallas_call(
        body,
        grid=(NB,),
        in_specs=[
            pl.BlockSpec((R, 1), lambda i: (i, 0)),
            pl.BlockSpec((NUM_ATOM, EMB), lambda i: (0, 0)),
            pl.BlockSpec((NUM_BOND, EMB), lambda i: (0, 0)),
        ],
        out_specs=[
            pl.BlockSpec((R, EMB), lambda i: (i, 0)),
            pl.BlockSpec((NUM_BOND, R, EMB), lambda i: (0, i, 0)),
        ],
        out_shape=[
            jax.ShapeDtypeStruct((N, EMB), jnp.float32),
            jax.ShapeDtypeStruct((NUM_BOND, N, EMB), jnp.float32),
        ],
    )(xf, x_emb, e_emb)


def _mlp_norm(h, aggr, W1, b1, W2, b2, gamma, beta, e_emb, last):
    def body(h_ref, a_ref, w1_ref, b1_ref, w2_ref, b2_ref, g_ref, be_ref,
             ee_ref, h_out, *rest):
        if last:
            (u_buf, st_buf) = rest
            t_out = None
        else:
            (t_out, u_buf, st_buf) = rest
        i = pl.program_id(0)

        @pl.when(i == 0)
        def _():
            st_buf[...] = jnp.zeros((8, EMB), jnp.float32)

        @pl.when(i < NB)
        def _():
            z = h_ref[...] + a_ref[...]
            t = jnp.maximum(
                jnp.dot(z, w1_ref[...], preferred_element_type=jnp.float32)
                + b1_ref[...], 0.0)
            u = jnp.dot(t, w2_ref[...],
                        preferred_element_type=jnp.float32) + b2_ref[...]
            u_buf[pl.ds(i * R, R), :] = u
            st_buf[0:1, :] += jnp.sum(u, axis=0, keepdims=True)
            st_buf[1:2, :] += jnp.sum(u * u, axis=0, keepdims=True)

        @pl.when(i >= NB)
        def _():
            j = i - NB
            mu = st_buf[0:1, :] * (1.0 / N)
            var = st_buf[1:2, :] * (1.0 / N) - mu * mu
            inv = lax.rsqrt(var + 1e-5)
            u = u_buf[pl.ds(j * R, R), :]
            hn = g_ref[...] * (u - mu) * inv + be_ref[...]
            if not last:
                hn = jnp.maximum(hn, 0.0)
            h_out[...] = hn
            if not last:
                for b in range(NUM_BOND):
                    t_out[b] = jnp.maximum(hn + ee_ref[b, :][None, :], 0.0)

    def blk(i):
        return jnp.where(i < NB, i, 0)

    def blk_out(i):
        return jnp.where(i >= NB, i - NB, 0)

    out_specs = [pl.BlockSpec((R, EMB), lambda i: (blk_out(i), 0))]
    out_shape = [jax.ShapeDtypeStruct((N, EMB), jnp.float32)]
    if not last:
        out_specs.append(
            pl.BlockSpec((NUM_BOND, R, EMB), lambda i: (0, blk_out(i), 0)))
        out_shape.append(
            jax.ShapeDtypeStruct((NUM_BOND, N, EMB), jnp.float32))

    res = pl.pallas_call(
        body,
        grid=(2 * NB,),
        in_specs=[
            pl.BlockSpec((R, EMB), lambda i: (blk(i), 0)),
            pl.BlockSpec((R, EMB), lambda i: (blk(i), 0)),
            pl.BlockSpec((EMB, 2 * EMB), lambda i: (0, 0)),
            pl.BlockSpec((1, 2 * EMB), lambda i: (0, 0)),
            pl.BlockSpec((2 * EMB, EMB), lambda i: (0, 0)),
            pl.BlockSpec((1, EMB), lambda i: (0, 0)),
            pl.BlockSpec((1, EMB), lambda i: (0, 0)),
            pl.BlockSpec((1, EMB), lambda i: (0, 0)),
            pl.BlockSpec((NUM_BOND, EMB), lambda i: (0, 0)),
        ],
        out_specs=out_specs,
        out_shape=out_shape,
        scratch_shapes=[
            pltpu.VMEM((N, EMB), jnp.float32),
            pltpu.VMEM((8, EMB), jnp.float32),
        ],
    )(h, aggr, W1, b1, W2, b2, gamma, beta, e_emb)
    if last:
        return res[0], None
    return res[0], res[1]


def _pool(h, batchf):
    def body(b_ref, h_ref, out_ref, acc_ref, cnt_ref):
        i = pl.program_id(0)

        @pl.when(i == 0)
        def _():
            acc_ref[...] = jnp.zeros((B, EMB), jnp.float32)
            cnt_ref[...] = jnp.zeros((B, EMB), jnp.float32)

        g = lax.broadcasted_iota(jnp.int32, (B, R), 0).astype(jnp.float32)
        m = (g == b_ref[0]).astype(jnp.float32)
        acc_ref[...] += jnp.dot(m, h_ref[...], preferred_element_type=jnp.float32)
        cnt_ref[...] += jnp.dot(m, jnp.ones((R, EMB), jnp.float32),
                                preferred_element_type=jnp.float32)

        @pl.when(i == NB - 1)
        def _():
            out_ref[...] = acc_ref[...] / jnp.maximum(cnt_ref[...], 1.0)

    return pl.pallas_call(
        body,
        grid=(NB,),
        in_specs=[
            pl.BlockSpec((1, 1, R), lambda i: (i, 0, 0)),
            pl.BlockSpec((R, EMB), lambda i: (i, 0)),
        ],
        out_specs=pl.BlockSpec((B, EMB), lambda i: (0, 0)),
        out_shape=jax.ShapeDtypeStruct((B, EMB), jnp.float32),
        scratch_shapes=[
            pltpu.VMEM((B, EMB), jnp.float32),
            pltpu.VMEM((B, EMB), jnp.float32),
        ],
    )(batchf, h)


def _head(pa, pb, context, Wo1, bo1, Wo2, bo2, Wc1, bc1, Wc2, bc2,
          Wm1, bm1, Wm2, bm2, Wm3, bm3):
    def body(pa_ref, pb_ref, c_ref, wo1, bo1r, wo2, bo2r, wc1, bc1r, wc2,
             bc2r, wm1, bm1r, wm2, bm2r, wm3, bm3r, o_ref):
        def mm(a, w, bias):
            return jnp.dot(a, w[...], preferred_element_type=jnp.float32) + bias[...]

        ha = mm(jnp.maximum(mm(pa_ref[...], wo1, bo1r), 0.0), wo2, bo2r)
        hb = mm(jnp.maximum(mm(pb_ref[...], wo1, bo1r), 0.0), wo2, bo2r)
        ctx = mm(jnp.maximum(mm(c_ref[...], wc1, bc1r), 0.0), wc2, bc2r)
        z = jnp.concatenate([ha, hb, ctx], axis=1)
        z = jnp.maximum(mm(z, wm1, bm1r), 0.0)
        z = jnp.maximum(mm(z, wm2, bm2r), 0.0)
        o_ref[...] = mm(z, wm3, bm3r)

    args = (pa, pb, context, Wo1, bo1, Wo2, bo2, Wc1, bc1, Wc2, bc2,
            Wm1, bm1, Wm2, bm2, Wm3, bm3)
    return pl.pallas_call(
        body,
        out_shape=jax.ShapeDtypeStruct((B, 1), jnp.float32),
    )(*args)


def kernel(xA, edge_indexA, edge_attrA, batchA, xB, edge_indexB, edge_attrB,
           batchB, context, params):
    pad_g = jnp.arange(K, dtype=jnp.int32)
    pad_d = jnp.full((K,), N, jnp.int32)

    def _prep(ei, ea):
        d = ei[1].astype(jnp.int32)
        order = jnp.argsort((d >= HN).astype(jnp.int32), stable=True)
        ds = d[order]
        gs = (ea[:, 0].astype(jnp.int32) * N + ei[0].astype(jnp.int32))[order]
        mid = jnp.sum((d < HN).astype(jnp.int32))
        st = jnp.concatenate([
            jnp.zeros((1,), jnp.int32), mid[None],
            jnp.full((22,), E, jnp.int32),
        ])
        ed = jnp.stack([jnp.concatenate([gs, pad_g]).reshape(NCH, K),
                        jnp.concatenate([ds, pad_d]).reshape(NCH, K)], axis=1)
        return ed, st

    edA, stA = _prep(edge_indexA, edge_attrA)
    edB, stB = _prep(edge_indexB, edge_attrB)
    zrows = jnp.zeros((PT_LAST + 8, EMB), jnp.float32)

    xfA = xA[:, 0].astype(jnp.float32)[:, None]
    xfB = xB[:, 0].astype(jnp.float32)[:, None]
    bfA = batchA.astype(jnp.float32).reshape(NB, 1, R)
    bfB = batchB.astype(jnp.float32).reshape(NB, 1, R)

    p = params
    row = lambda v: v[None, :]

    hA, TA = _embed(xfA, p['x_emb'], p['e_emb'])
    hB, TB = _embed(xfB, p['x_emb'], p['e_emb'])
    for l in range(NUM_LAYER):
        gl = p['gnn'][l]
        aggrA = _sc_aggregate(TA.reshape(NUM_BOND * N, EMB), edA, stA,
                              zrows).reshape(N, EMB)
        aggrB = _sc_aggregate(TB.reshape(NUM_BOND * N, EMB), edB, stB,
                              zrows).reshape(N, EMB)
        args = (gl['W1'], row(gl['b1']), gl['W2'], row(gl['b2']),
                row(gl['gamma']), row(gl['beta']), p['e_emb'])
        hA, TA = _mlp_norm(hA, aggrA, *args, last=(l == NUM_LAYER - 1))
        hB, TB = _mlp_norm(hB, aggrB, *args, last=(l == NUM_LAYER - 1))

    pa = _pool(hA, bfA)
    pb = _pool(hB, bfB)
    return _head(pa, pb, context,
                 p['Wo1'], row(p['bo1']), p['Wo2'], row(p['bo2']),
                 p['Wc1'], row(p['bc1']), p['Wc2'], row(p['bc2']),
                 p['Wm1'], row(p['bm1']), p['Wm2'], row(p['bm2']),
                 p['Wm3'], row(p['bm3']))


# speedup vs baseline: 6.0160x; 1.0034x over previous
import functools

import jax
import jax.numpy as jnp
from jax import lax
from jax.experimental import pallas as pl
from jax.experimental.pallas import tpu as pltpu
from jax.experimental.pallas import tpu_sc as plsc

N = 10000
E = 160000
B = 64
EMB = 128
NUM_BOND = 6
NUM_ATOM = 120
NUM_LAYER = 5

R = 2000
NB = N // R

SC_SUB = 16
K = 128
HN = N // 2
PT = HN // SC_SUB // 8 * 8
PT_LAST = HN - 15 * PT
ACC_ROWS = HN + 8
EPAD = E + K
NCH = EPAD // K


def _sc_aggregate(T2, edata, starts, zrows):
    mesh = plsc.VectorSubcoreMesh(core_axis_name="c", subcore_axis_name="s")

    @functools.partial(
        pl.kernel,
        out_type=jax.ShapeDtypeStruct((2, HN, EMB), jnp.float32),
        mesh=mesh,
        scratch_types=[
            pltpu.VMEM((3, 2, K), jnp.int32),
            pltpu.VMEM((3, K), jnp.int32),
            pltpu.VMEM((3, K, EMB), jnp.float32),
            pltpu.VMEM((24,), jnp.int32),
            pltpu.VMEM_SHARED((ACC_ROWS, EMB), jnp.float32),
            pltpu.SemaphoreType.DMA,
            pltpu.SemaphoreType.DMA,
            pltpu.SemaphoreType.DMA,
            pltpu.SemaphoreType.DMA,
            pltpu.SemaphoreType.DMA,
            pltpu.SemaphoreType.DMA,
            pltpu.SemaphoreType.DMA,
            pltpu.SemaphoreType.DMA,
            pltpu.SemaphoreType.DMA,
        ],
    )
    def agg(t_hbm, ed_hbm, st_hbm, z_hbm, out_hbm,
            ed_v, dl_v, msg_v, st_v, acc_sh,
            sed0, sed1, sed2, sg0, sg1, sg2, ss0, ss1, ss2):
        c = lax.axis_index("c")
        s = lax.axis_index("s")
        sed = (sed0, sed1, sed2)
        sg = (sg0, sg1, sg2)
        ss = (ss0, ss1, ss2)

        pltpu.sync_copy(st_hbm, st_v)
        svec = st_v[pl.ds(c, 16)]
        trash = HN + (jnp.arange(16, dtype=jnp.int32) & 7)
        row0 = s * PT

        @pl.when(s < SC_SUB - 1)
        def _():
            pltpu.sync_copy(z_hbm.at[pl.ds(0, PT)], acc_sh.at[pl.ds(row0, PT)])

        @pl.when(s == SC_SUB - 1)
        def _():
            pltpu.sync_copy(z_hbm, acc_sh.at[pl.ds(15 * PT, PT_LAST + 8)])

        plsc.subcore_barrier()

        s0 = svec[0]
        s1 = svec[1]
        clo = s0 // K
        chi = (s1 + K - 1) // K
        nch = chi - clo
        nmy = (nch - s + SC_SUB - 1) // SC_SUB
        cbase = clo + s
        pbase = c * HN

        def chunk(i):
            return ed_hbm.at[cbase + i * SC_SUB]

        def fetch(i, b):
            return pltpu.async_copy(chunk(i), ed_v.at[b], sed[b])

        def remap(b):
            for j in range(K // 16):
                d = ed_v[b, 1, j * 16:(j + 1) * 16] - pbase
                ok = (d >= 0) & (d < HN)
                dl_v[b, j * 16:(j + 1) * 16] = jnp.where(ok, d, trash)

        def gather(i, b):
            return pltpu.async_copy(t_hbm.at[ed_v.at[b, 0]], msg_v.at[b],
                                    sg[b])

        def wait_scatter(b):
            pltpu.make_async_copy(msg_v.at[b], acc_sh.at[dl_v.at[b]],
                                  ss[b]).wait()

        @pl.when(nmy > 0)
        def _():
            fetch(0, 0)
            pltpu.make_async_copy(chunk(0), ed_v.at[0], sed[0]).wait()
            gather(0, 0)
            remap(0)

        @pl.when(nmy > 1)
        def _():
            fetch(1, 1)

        def outer(i, _):
            for q in range(3):
                tt = 3 * i + q
                b = q
                bn = (q + 1) % 3
                b2 = (q + 2) % 3

                @pl.when(tt < nmy)
                def _():
                    @pl.when(tt + 1 < nmy)
                    def _():
                        pltpu.make_async_copy(chunk(tt + 1), ed_v.at[bn],
                                              sed[bn]).wait()

                        @pl.when(tt >= 2)
                        def _():
                            wait_scatter(bn)

                        gather(tt + 1, bn)
                        remap(bn)

                        @pl.when(tt + 2 < nmy)
                        def _():
                            fetch(tt + 2, b2)

                    pltpu.make_async_copy(t_hbm.at[ed_v.at[b, 0]],
                                          msg_v.at[b], sg[b]).wait()
                    pltpu.async_copy(msg_v.at[b], acc_sh.at[dl_v.at[b]],
                                     ss[b], add=True)
            return 0

        lax.fori_loop(0, (nmy + 2) // 3, outer, 0)

        for b in range(3):
            @pl.when((((nmy - 1) % 3 == b) & (nmy >= 1))
                     | (((nmy - 2) % 3 == b) & (nmy >= 2))
                     | (((nmy - 3) % 3 == b) & (nmy >= 3)))
            def _():
                wait_scatter(b)

        plsc.subcore_barrier()

        @pl.when(s < SC_SUB - 1)
        def _():
            pltpu.sync_copy(acc_sh.at[pl.ds(row0, PT)],
                            out_hbm.at[c].at[pl.ds(row0, PT)])

        @pl.when(s == SC_SUB - 1)
        def _():
            pltpu.sync_copy(acc_sh.at[pl.ds(15 * PT, PT_LAST)],
                            out_hbm.at[c].at[pl.ds(15 * PT, PT_LAST)])

    return agg(T2, edata, starts, zrows)


def _embed(xf, x_emb, e_emb):
    def body(x_ref, xe_ref, ee_ref, h_ref, t_ref):
        ids = x_ref[...]
        io = lax.broadcasted_iota(jnp.int32, (R, NUM_ATOM), 1).astype(jnp.float32)
        oh = (io == ids).astype(jnp.float32)
        h = jnp.dot(oh, xe_ref[...], preferred_element_type=jnp.float32)
        h_ref[...] = h
        for b in range(NUM_BOND):
            t_ref[b] = jnp.maximum(h + ee_ref[b, :][None, :], 0.0)

    return pl.pallas_call(
        body,
        grid=(NB,),
        in_specs=[
            pl.BlockSpec((R, 1), lambda i: (i, 0)),
            pl.BlockSpec((NUM_ATOM, EMB), lambda i: (0, 0)),
            pl.BlockSpec((NUM_BOND, EMB), lambda i: (0, 0)),
        ],
        out_specs=[
            pl.BlockSpec((R, EMB), lambda i: (i, 0)),
            pl.BlockSpec((NUM_BOND, R, EMB), lambda i: (0, i, 0)),
        ],
        out_shape=[
            jax.ShapeDtypeStruct((N, EMB), jnp.float32),
            jax.ShapeDtypeStruct((NUM_BOND, N, EMB), jnp.float32),
        ],
    )(xf, x_emb, e_emb)


def _mlp_norm(h, aggr, W1, b1, W2, b2, gamma, beta, e_emb, batchf, last):
    def body(h_ref, a_ref, w1_ref, b1_ref, w2_ref, b2_ref, g_ref, be_ref,
             ee_ref, *rest):
        if last:
            (bf_ref, p_out, u_buf, st_buf, acc_b, cnt_b) = rest
        else:
            (h_out, t_out, u_buf, st_buf) = rest
        i = pl.program_id(0)

        @pl.when(i == 0)
        def _():
            st_buf[...] = jnp.zeros((8, EMB), jnp.float32)
            if last:
                acc_b[...] = jnp.zeros((B, EMB), jnp.float32)
                cnt_b[...] = jnp.zeros((B, EMB), jnp.float32)

        @pl.when(i < NB)
        def _():
            z = h_ref[...] + a_ref[...]
            t = jnp.maximum(
                jnp.dot(z, w1_ref[...], preferred_element_type=jnp.float32)
                + b1_ref[...], 0.0)
            u = jnp.dot(t, w2_ref[...],
                        preferred_element_type=jnp.float32) + b2_ref[...]
            u_buf[pl.ds(i * R, R), :] = u
            st_buf[0:1, :] += jnp.sum(u, axis=0, keepdims=True)
            st_buf[1:2, :] += jnp.sum(u * u, axis=0, keepdims=True)

        @pl.when(i >= NB)
        def _():
            j = i - NB
            mu = st_buf[0:1, :] * (1.0 / N)
            var = st_buf[1:2, :] * (1.0 / N) - mu * mu
            inv = lax.rsqrt(var + 1e-5)
            u = u_buf[pl.ds(j * R, R), :]
            hn = g_ref[...] * (u - mu) * inv + be_ref[...]
            if not last:
                hn = jnp.maximum(hn, 0.0)
                h_out[...] = hn
                for b in range(NUM_BOND):
                    t_out[b] = jnp.maximum(hn + ee_ref[b, :][None, :], 0.0)
            else:
                g = lax.broadcasted_iota(jnp.int32, (B, R), 0).astype(
                    jnp.float32)
                m = (g == bf_ref[0]).astype(jnp.float32)
                acc_b[...] += jnp.dot(m, hn,
                                      preferred_element_type=jnp.float32)
                cnt_b[...] += jnp.dot(m, jnp.ones((R, EMB), jnp.float32),
                                      preferred_element_type=jnp.float32)

                @pl.when(i == 2 * NB - 1)
                def _():
                    p_out[...] = acc_b[...] / jnp.maximum(cnt_b[...], 1.0)

    def blk(i):
        return jnp.where(i < NB, i, 0)

    def blk_out(i):
        return jnp.where(i >= NB, i - NB, 0)

    in_specs = [
        pl.BlockSpec((R, EMB), lambda i: (blk(i), 0)),
        pl.BlockSpec((R, EMB), lambda i: (blk(i), 0)),
        pl.BlockSpec((EMB, 2 * EMB), lambda i: (0, 0)),
        pl.BlockSpec((1, 2 * EMB), lambda i: (0, 0)),
        pl.BlockSpec((2 * EMB, EMB), lambda i: (0, 0)),
        pl.BlockSpec((1, EMB), lambda i: (0, 0)),
        pl.BlockSpec((1, EMB), lambda i: (0, 0)),
        pl.BlockSpec((1, EMB), lambda i: (0, 0)),
        pl.BlockSpec((NUM_BOND, EMB), lambda i: (0, 0)),
    ]
    args = [h, aggr, W1, b1, W2, b2, gamma, beta, e_emb]
    scratch = [
        pltpu.VMEM((N, EMB), jnp.float32),
        pltpu.VMEM((8, EMB), jnp.float32),
    ]
    if last:
        in_specs.append(pl.BlockSpec((1, 1, R), lambda i: (blk_out(i), 0, 0)))
        args.append(batchf)
        out_specs = pl.BlockSpec((B, EMB), lambda i: (0, 0))
        out_shape = jax.ShapeDtypeStruct((B, EMB), jnp.float32)
        scratch += [pltpu.VMEM((B, EMB), jnp.float32),
                    pltpu.VMEM((B, EMB), jnp.float32)]
    else:
        out_specs = [
            pl.BlockSpec((R, EMB), lambda i: (blk_out(i), 0)),
            pl.BlockSpec((NUM_BOND, R, EMB), lambda i: (0, blk_out(i), 0)),
        ]
        out_shape = [
            jax.ShapeDtypeStruct((N, EMB), jnp.float32),
            jax.ShapeDtypeStruct((NUM_BOND, N, EMB), jnp.float32),
        ]

    res = pl.pallas_call(
        body,
        grid=(2 * NB,),
        in_specs=in_specs,
        out_specs=out_specs,
        out_shape=out_shape,
        scratch_shapes=scratch,
    )(*args)
    if last:
        return res, None
    return res[0], res[1]


def _head(pa, pb, context, Wo1, bo1, Wo2, bo2, Wc1, bc1, Wc2, bc2,
          Wm1, bm1, Wm2, bm2, Wm3, bm3):
    def body(pa_ref, pb_ref, c_ref, wo1, bo1r, wo2, bo2r, wc1, bc1r, wc2,
             bc2r, wm1, bm1r, wm2, bm2r, wm3, bm3r, o_ref):
        def mm(a, w, bias):
            return jnp.dot(a, w[...], preferred_element_type=jnp.float32) + bias[...]

        ha = mm(jnp.maximum(mm(pa_ref[...], wo1, bo1r), 0.0), wo2, bo2r)
        hb = mm(jnp.maximum(mm(pb_ref[...], wo1, bo1r), 0.0), wo2, bo2r)
        ctx = mm(jnp.maximum(mm(c_ref[...], wc1, bc1r), 0.0), wc2, bc2r)
        z = jnp.concatenate([ha, hb, ctx], axis=1)
        z = jnp.maximum(mm(z, wm1, bm1r), 0.0)
        z = jnp.maximum(mm(z, wm2, bm2r), 0.0)
        o_ref[...] = mm(z, wm3, bm3r)

    args = (pa, pb, context, Wo1, bo1, Wo2, bo2, Wc1, bc1, Wc2, bc2,
            Wm1, bm1, Wm2, bm2, Wm3, bm3)
    return pl.pallas_call(
        body,
        out_shape=jax.ShapeDtypeStruct((B, 1), jnp.float32),
    )(*args)


def kernel(xA, edge_indexA, edge_attrA, batchA, xB, edge_indexB, edge_attrB,
           batchB, context, params):
    pad_g = jnp.arange(K, dtype=jnp.int32)
    pad_d = jnp.full((K,), N, jnp.int32)

    def _prep(ei, ea):
        d = ei[1].astype(jnp.int32)
        order = jnp.argsort((d >= HN).astype(jnp.int32), stable=True)
        ds = d[order]
        gs = (ea[:, 0].astype(jnp.int32) * N + ei[0].astype(jnp.int32))[order]
        mid = jnp.sum((d < HN).astype(jnp.int32))
        st = jnp.concatenate([
            jnp.zeros((1,), jnp.int32), mid[None],
            jnp.full((22,), E, jnp.int32),
        ])
        ed = jnp.stack([jnp.concatenate([gs, pad_g]).reshape(NCH, K),
                        jnp.concatenate([ds, pad_d]).reshape(NCH, K)], axis=1)
        return ed, st

    edA, stA = _prep(edge_indexA, edge_attrA)
    edB, stB = _prep(edge_indexB, edge_attrB)
    zrows = jnp.zeros((PT_LAST + 8, EMB), jnp.float32)

    xfA = xA[:, 0].astype(jnp.float32)[:, None]
    xfB = xB[:, 0].astype(jnp.float32)[:, None]
    bfA = batchA.astype(jnp.float32).reshape(NB, 1, R)
    bfB = batchB.astype(jnp.float32).reshape(NB, 1, R)

    p = params
    row = lambda v: v[None, :]

    hA, TA = _embed(xfA, p['x_emb'], p['e_emb'])
    hB, TB = _embed(xfB, p['x_emb'], p['e_emb'])
    for l in range(NUM_LAYER):
        gl = p['gnn'][l]
        aggrA = _sc_aggregate(TA.reshape(NUM_BOND * N, EMB), edA, stA,
                              zrows).reshape(N, EMB)
        aggrB = _sc_aggregate(TB.reshape(NUM_BOND * N, EMB), edB, stB,
                              zrows).reshape(N, EMB)
        args = (gl['W1'], row(gl['b1']), gl['W2'], row(gl['b2']),
                row(gl['gamma']), row(gl['beta']), p['e_emb'])
        hA, TA = _mlp_norm(hA, aggrA, *args, bfA, last=(l == NUM_LAYER - 1))
        hB, TB = _mlp_norm(hB, aggrB, *args, bfB, last=(l == NUM_LAYER - 1))

    pa, pb = hA, hB
    return _head(pa, pb, context,
                 p['Wo1'], row(p['bo1']), p['Wo2'], row(p['bo2']),
                 p['Wc1'], row(p['bc1']), p['Wc2'], row(p['bc2']),
                 p['Wm1'], row(p['bm1']), p['Wm2'], row(p['bm2']),
                 p['Wm3'], row(p['bm3']))
